# baseline (speedup 1.0000x reference)
import functools

import jax
import jax.numpy as jnp
from jax import lax
from jax.experimental import pallas as pl
from jax.experimental.pallas import tpu as pltpu
from jax.experimental.pallas import tpu_sc as plsc

NUM_ENTITIES = 100000
NUM_RELATIONS = 1000
DIM = 200
BATCH = 16384

NC, NS, L = 2, 16, 16
NW = NC * NS
B_PER_W = BATCH // NW
CHUNK = 32
NCHUNK = B_PER_W // CHUNK
NFULL = DIM // L
TAIL_OFF = DIM - L
TAIL_KEEP = L - (DIM - NFULL * L)
ACC_STRIDE = L + 1


def _score_chunk(rh, rt, rr, accbuf, out_v, out_base, tail_mask, lane):
    def body(i, carry):
        acc = jnp.zeros((L,), jnp.float32)
        for j in range(NFULL + 1):
            off = j * L if j < NFULL else TAIL_OFF
            h_re = rh[i, pl.ds(off, L)]
            h_im = rh[i, pl.ds(off + DIM, L)]
            t_re = rt[i, pl.ds(off, L)]
            t_im = rt[i, pl.ds(off + DIM, L)]
            r_re = rr[i, pl.ds(off, L)]
            r_im = rr[i, pl.ds(off + DIM, L)]
            p = h_re * t_re + h_im * t_im
            q = h_re * t_im - h_im * t_re
            term = r_re * p + r_im * q
            if j == NFULL:
                term = jnp.where(tail_mask, term, 0.0)
            acc = acc + term
        accbuf[pl.ds(i * ACC_STRIDE, L)] = acc
        return carry

    lax.fori_loop(0, CHUNK, body, 0)
    for k in range(0, CHUNK, L):
        base_idx = (lane + k) * ACC_STRIDE
        tot = jnp.zeros((L,), jnp.float32)
        for j in range(L):
            tot = tot + plsc.load_gather(accbuf, [base_idx + j])
        out_v[pl.ds(out_base + k, L)] = tot


def _complex_score_kernel(heads_hbm, rels_hbm, tails_hbm, ent_hbm, rel_hbm,
                          out_hbm, idx_h, idx_t, idx_r, rows_h, rows_t,
                          rows_r, accbuf, out_v, sem):
    wid = lax.axis_index("s") * NC + lax.axis_index("c")
    base = wid * B_PER_W
    lane = lax.iota(jnp.int32, L)
    tail_mask = lane >= TAIL_KEEP

    def chunk_body(g, carry):
        cbase = base + g * CHUNK
        pltpu.sync_copy(heads_hbm.at[pl.ds(cbase, CHUNK)], idx_h)
        pltpu.sync_copy(tails_hbm.at[pl.ds(cbase, CHUNK)], idx_t)
        pltpu.sync_copy(rels_hbm.at[pl.ds(cbase, CHUNK)], idx_r)
        c1 = pltpu.async_copy(ent_hbm.at[idx_h], rows_h, sem)
        c2 = pltpu.async_copy(ent_hbm.at[idx_t], rows_t, sem)
        c3 = pltpu.async_copy(rel_hbm.at[idx_r], rows_r, sem)
        c1.wait()
        c2.wait()
        c3.wait()
        _score_chunk(rows_h, rows_t, rows_r, accbuf, out_v, g * CHUNK,
                     tail_mask, lane)
        return carry

    lax.fori_loop(0, NCHUNK, chunk_body, 0)
    pltpu.sync_copy(out_v, out_hbm.at[pl.ds(base, B_PER_W)])


@jax.jit
def _compl_ex(heads, rels, tails, entity_emb, rel_emb):
    mesh = plsc.VectorSubcoreMesh(
        core_axis_name="c", subcore_axis_name="s", num_cores=NC,
        num_subcores=NS)
    run = functools.partial(
        pl.kernel,
        out_type=jax.ShapeDtypeStruct((BATCH,), jnp.float32),
        mesh=mesh,
        compiler_params=pltpu.CompilerParams(
            needs_layout_passes=False, use_tc_tiling_on_sc=False),
        scratch_types=[
            pltpu.VMEM((CHUNK,), jnp.int32),
            pltpu.VMEM((CHUNK,), jnp.int32),
            pltpu.VMEM((CHUNK,), jnp.int32),
            pltpu.VMEM((CHUNK, 2 * DIM), jnp.float32),
            pltpu.VMEM((CHUNK, 2 * DIM), jnp.float32),
            pltpu.VMEM((CHUNK, 2 * DIM), jnp.float32),
            pltpu.VMEM((CHUNK * ACC_STRIDE,), jnp.float32),
            pltpu.VMEM((B_PER_W,), jnp.float32),
            pltpu.SemaphoreType.DMA,
        ],
    )(_complex_score_kernel)
    return run(heads, rels, tails, entity_emb, rel_emb)


def kernel(heads, rels, tails, entity_emb, rel_emb):
    zero = lax.optimization_barrier(jnp.float32(0.0))
    return _compl_ex(
        heads.astype(jnp.int32),
        rels.astype(jnp.int32),
        tails.astype(jnp.int32),
        entity_emb.astype(jnp.float32) + zero,
        rel_emb.astype(jnp.float32) + zero,
    )


# speedup vs baseline: 3.7183x; 3.7183x over previous
import functools

import jax
import jax.numpy as jnp
from jax import lax
from jax.experimental import pallas as pl
from jax.experimental.pallas import tpu as pltpu
from jax.experimental.pallas import tpu_sc as plsc

NUM_ENTITIES = 100000
NUM_RELATIONS = 1000
DIM = 200
BATCH = 16384

NC, NS, L = 2, 16, 16
NW = NC * NS
B_PER_W = BATCH // NW
CHUNK = 32
NCHUNK = B_PER_W // CHUNK
ACC_STRIDE = L + 1
TAIL = 3 * 128

_GROUPS = [
    (0, 0, 1, 72, False), (0, 16, 1, 88, False), (0, 32, 1, 104, False),
    (0, 40, 1, 112, True),
    (0, 56, 2, 0, False), (0, 72, 2, 16, False), (0, 88, 2, 32, False),
    (0, 104, 2, 48, False), (0, 112, 2, 56, True),
    (1, 0, 2, 72, False), (1, 16, 2, 88, False), (1, 32, 2, 104, False),
    (1, 40, 2, 112, True),
    (1, 56, 3, 0, False),
]


def _score_chunk(bufs, accbuf, out_v, out_base, half_mask, lane):
    def body(i, carry):
        acc = jnp.zeros((L,), jnp.float32)
        for re_p, re_o, im_p, im_o, masked in _GROUPS:
            h_re = bufs[0][re_p][i, pl.ds(re_o, L)]
            h_im = bufs[0][im_p][i, pl.ds(im_o, L)]
            t_re = bufs[1][re_p][i, pl.ds(re_o, L)]
            t_im = bufs[1][im_p][i, pl.ds(im_o, L)]
            r_re = bufs[2][re_p][i, pl.ds(re_o, L)]
            r_im = bufs[2][im_p][i, pl.ds(im_o, L)]
            p = h_re * t_re + h_im * t_im
            q = h_re * t_im - h_im * t_re
            term = r_re * p + r_im * q
            if masked:
                term = jnp.where(half_mask, term, 0.0)
            acc = acc + term
        accbuf[pl.ds(i * ACC_STRIDE, L)] = acc
        return carry

    lax.fori_loop(0, CHUNK, body, 0)
    for k in range(0, CHUNK, L):
        base_idx = (lane + k) * ACC_STRIDE
        tot = jnp.zeros((L,), jnp.float32)
        for j in range(L):
            tot = tot + plsc.load_gather(accbuf, [base_idx + j])
        out_v[pl.ds(out_base + k, L)] = tot


def _complex_score_kernel(heads_hbm, rels_hbm, tails_hbm, ent_hbm, rel_hbm,
                          ent_tail_hbm, rel_tail_hbm, out_hbm,
                          idx_h, idx_t, idx_r, bufs_flat, accbuf, out_v, sem):
    wid = lax.axis_index("s") * NC + lax.axis_index("c")
    base = wid * B_PER_W
    lane = lax.iota(jnp.int32, L)
    half_mask = lane >= L // 2
    bufs = [bufs_flat[4 * t:4 * t + 4] for t in range(3)]

    def chunk_body(g, carry):
        cbase = base + g * CHUNK
        pltpu.sync_copy(heads_hbm.at[pl.ds(cbase, CHUNK)], idx_h)
        pltpu.sync_copy(tails_hbm.at[pl.ds(cbase, CHUNK)], idx_t)
        pltpu.sync_copy(rels_hbm.at[pl.ds(cbase, CHUNK)], idx_r)
        copies = []
        for idx, table, tail_table, tb in (
                (idx_h, ent_hbm, ent_tail_hbm, bufs[0]),
                (idx_t, ent_hbm, ent_tail_hbm, bufs[1]),
                (idx_r, rel_hbm, rel_tail_hbm, bufs[2])):
            for k in range(3):
                copies.append(pltpu.async_copy(
                    table.at[idx, pl.ds(128 * k, 128)], tb[k], sem))
            copies.append(pltpu.async_copy(tail_table.at[idx], tb[3], sem))
        for c in copies:
            c.wait()
        _score_chunk(bufs, accbuf, out_v, g * CHUNK, half_mask, lane)
        return carry

    lax.fori_loop(0, NCHUNK, chunk_body, 0)
    pltpu.sync_copy(out_v, out_hbm.at[pl.ds(base, B_PER_W)])


@jax.jit
def _compl_ex(heads, rels, tails, entity_emb, rel_emb):
    mesh = plsc.VectorSubcoreMesh(
        core_axis_name="c", subcore_axis_name="s", num_cores=NC,
        num_subcores=NS)
    ent_tail = jnp.pad(entity_emb[:, TAIL:], ((0, 0), (0, 112)))
    rel_tail = jnp.pad(rel_emb[:, TAIL:], ((0, 0), (0, 112)))
    run = functools.partial(
        pl.kernel,
        out_type=jax.ShapeDtypeStruct((BATCH,), jnp.float32),
        mesh=mesh,
        compiler_params=pltpu.CompilerParams(needs_layout_passes=False),
        scratch_types=[
            pltpu.VMEM((CHUNK,), jnp.int32),
            pltpu.VMEM((CHUNK,), jnp.int32),
            pltpu.VMEM((CHUNK,), jnp.int32),
            [pltpu.VMEM((CHUNK, 128), jnp.float32) for _ in range(12)],
            pltpu.VMEM((CHUNK * ACC_STRIDE,), jnp.float32),
            pltpu.VMEM((B_PER_W,), jnp.float32),
            pltpu.SemaphoreType.DMA,
        ],
    )(_complex_score_kernel)
    return run(heads, rels, tails, entity_emb, rel_emb, ent_tail, rel_tail)


def kernel(heads, rels, tails, entity_emb, rel_emb):
    return _compl_ex(
        heads.astype(jnp.int32),
        rels.astype(jnp.int32),
        tails.astype(jnp.int32),
        entity_emb.astype(jnp.float32),
        rel_emb.astype(jnp.float32),
    )


# speedup vs baseline: 3.9830x; 1.0712x over previous
import functools

import jax
import jax.numpy as jnp
from jax import lax
from jax.experimental import pallas as pl
from jax.experimental.pallas import tpu as pltpu
from jax.experimental.pallas import tpu_sc as plsc

NUM_ENTITIES = 100000
NUM_RELATIONS = 1000
DIM = 200
BATCH = 16384

NC, NS, L = 2, 16, 16
NW = NC * NS
B_PER_W = BATCH // NW
CHUNK = 32
NCHUNK = B_PER_W // CHUNK
ACC_STRIDE = L + 1
TAIL = 3 * 128

_GROUPS = [
    (0, 0, 1, 72, False), (0, 16, 1, 88, False), (0, 32, 1, 104, False),
    (0, 40, 1, 112, True),
    (0, 56, 2, 0, False), (0, 72, 2, 16, False), (0, 88, 2, 32, False),
    (0, 104, 2, 48, False), (0, 112, 2, 56, True),
    (1, 0, 2, 72, False), (1, 16, 2, 88, False), (1, 32, 2, 104, False),
    (1, 40, 2, 112, True),
    (1, 56, 3, 0, False),
]


def _score_chunk(bufs, idxs, accbuf, out_v, out_base, half_mask, lane):
    def body(i, carry):
        acc = jnp.zeros((L,), jnp.float32)
        offs = [(idxs[t][pl.ds(i, L)][0] & 7) * L for t in range(3)]
        for re_p, re_o, im_p, im_o, masked in _GROUPS:
            if im_p == 3:
                h_im = bufs[0][3][i, pl.ds(offs[0], L)]
                t_im = bufs[1][3][i, pl.ds(offs[1], L)]
                r_im = bufs[2][3][i, pl.ds(offs[2], L)]
            else:
                h_im = bufs[0][im_p][i, pl.ds(im_o, L)]
                t_im = bufs[1][im_p][i, pl.ds(im_o, L)]
                r_im = bufs[2][im_p][i, pl.ds(im_o, L)]
            h_re = bufs[0][re_p][i, pl.ds(re_o, L)]
            t_re = bufs[1][re_p][i, pl.ds(re_o, L)]
            r_re = bufs[2][re_p][i, pl.ds(re_o, L)]
            p = h_re * t_re + h_im * t_im
            q = h_re * t_im - h_im * t_re
            term = r_re * p + r_im * q
            if masked:
                term = jnp.where(half_mask, term, 0.0)
            acc = acc + term
        accbuf[pl.ds(i * ACC_STRIDE, L)] = acc
        return carry

    lax.fori_loop(0, CHUNK, body, 0)
    for k in range(0, CHUNK, L):
        base_idx = (lane + k) * ACC_STRIDE
        tot = jnp.zeros((L,), jnp.float32)
        for j in range(L):
            tot = tot + plsc.load_gather(accbuf, [base_idx + j])
        out_v[pl.ds(out_base + k, L)] = tot


def _complex_score_kernel(heads_hbm, rels_hbm, tails_hbm, ent_hbm, rel_hbm,
                          ent_tail_hbm, rel_tail_hbm, out_hbm,
                          idx_h, idx_t, idx_r, idx_ht, idx_tt, idx_rt,
                          pad_h, pad_t, pad_r, bufs_flat, accbuf, out_v, sem):
    wid = lax.axis_index("s") * NC + lax.axis_index("c")
    base = wid * B_PER_W
    lane = lax.iota(jnp.int32, L)
    half_mask = lane >= L // 2
    bufs = [bufs_flat[4 * t:4 * t + 4] for t in range(3)]
    idxs = (idx_h, idx_t, idx_r)
    tidxs = (idx_ht, idx_tt, idx_rt)
    pads = (pad_h, pad_t, pad_r)

    def chunk_body(g, carry):
        cbase = base + g * CHUNK
        pltpu.sync_copy(heads_hbm.at[pl.ds(cbase, CHUNK)], idx_h)
        pltpu.sync_copy(tails_hbm.at[pl.ds(cbase, CHUNK)], idx_t)
        pltpu.sync_copy(rels_hbm.at[pl.ds(cbase, CHUNK)], idx_r)
        for t in range(3):
            for c in range(0, CHUNK, L):
                v = idxs[t][pl.ds(c, L)]
                tidxs[t][pl.ds(c, L)] = v >> 3
                pads[t][pl.ds(c, L)] = v
        copies = []
        for t, (table, tail_table) in enumerate(
                ((ent_hbm, ent_tail_hbm), (ent_hbm, ent_tail_hbm),
                 (rel_hbm, rel_tail_hbm))):
            for k in range(3):
                copies.append(pltpu.async_copy(
                    table.at[idxs[t], pl.ds(128 * k, 128)], bufs[t][k], sem))
            copies.append(pltpu.async_copy(
                tail_table.at[tidxs[t]], bufs[t][3], sem))
        for c in copies:
            c.wait()
        _score_chunk(bufs, pads, accbuf, out_v, g * CHUNK, half_mask, lane)
        return carry

    lax.fori_loop(0, NCHUNK, chunk_body, 0)
    pltpu.sync_copy(out_v, out_hbm.at[pl.ds(base, B_PER_W)])


@jax.jit
def _compl_ex(heads, rels, tails, entity_emb, rel_emb):
    mesh = plsc.VectorSubcoreMesh(
        core_axis_name="c", subcore_axis_name="s", num_cores=NC,
        num_subcores=NS)
    ent_tail = entity_emb[:, TAIL:].reshape(NUM_ENTITIES // 8, 128)
    rel_tail = rel_emb[:, TAIL:].reshape(NUM_RELATIONS // 8, 128)
    run = functools.partial(
        pl.kernel,
        out_type=jax.ShapeDtypeStruct((BATCH,), jnp.float32),
        mesh=mesh,
        compiler_params=pltpu.CompilerParams(needs_layout_passes=False),
        scratch_types=[
            pltpu.VMEM((CHUNK,), jnp.int32),
            pltpu.VMEM((CHUNK,), jnp.int32),
            pltpu.VMEM((CHUNK,), jnp.int32),
            pltpu.VMEM((CHUNK,), jnp.int32),
            pltpu.VMEM((CHUNK,), jnp.int32),
            pltpu.VMEM((CHUNK,), jnp.int32),
            pltpu.VMEM((CHUNK + L,), jnp.int32),
            pltpu.VMEM((CHUNK + L,), jnp.int32),
            pltpu.VMEM((CHUNK + L,), jnp.int32),
            [pltpu.VMEM((CHUNK, 128), jnp.float32) for _ in range(12)],
            pltpu.VMEM((CHUNK * ACC_STRIDE,), jnp.float32),
            pltpu.VMEM((B_PER_W,), jnp.float32),
            pltpu.SemaphoreType.DMA,
        ],
    )(_complex_score_kernel)
    return run(heads, rels, tails, entity_emb, rel_emb, ent_tail, rel_tail)


def kernel(heads, rels, tails, entity_emb, rel_emb):
    return _compl_ex(
        heads.astype(jnp.int32),
        rels.astype(jnp.int32),
        tails.astype(jnp.int32),
        entity_emb.astype(jnp.float32),
        rel_emb.astype(jnp.float32),
    )


# speedup vs baseline: 4.0226x; 1.0099x over previous
import functools

import jax
import jax.numpy as jnp
from jax import lax
from jax.experimental import pallas as pl
from jax.experimental.pallas import tpu as pltpu
from jax.experimental.pallas import tpu_sc as plsc

NUM_ENTITIES = 100000
NUM_RELATIONS = 1000
DIM = 200
BATCH = 16384

NC, NS, L = 2, 16, 16
NW = NC * NS
B_PER_W = BATCH // NW
CHUNK = 32
NCHUNK = B_PER_W // CHUNK
ACC_STRIDE = L + 1
SEG = 184

_GROUPS = [
    (0, 0, 1, 72, False), (0, 16, 1, 88, False), (0, 32, 1, 104, False),
    (0, 40, 1, 112, True),
    (0, 56, 2, 0, False), (0, 72, 2, 16, False), (0, 88, 2, 32, False),
    (0, 104, 2, 48, False), (0, 112, 2, 56, True),
    (1, 0, 2, 72, False), (1, 16, 2, 88, False), (1, 32, 2, 104, False),
    (1, 40, 2, 112, True),
]


def _transpose_reduce(accbuf, out_v, out_base, lane, count):
    for k in range(0, count, L):
        base_idx = (lane + k) * ACC_STRIDE
        tot = jnp.zeros((L,), jnp.float32)
        for j in range(L):
            tot = tot + plsc.load_gather(accbuf, [base_idx + j])
        out_v[pl.ds(out_base + k, L)] = tot


def _main_kernel(heads_hbm, rels_hbm, tails_hbm, ent_hbm, rel_hbm,
                 out_hbm, idx_h, idx_t, idx_r, bufs_flat, accbuf, out_v,
                 sem):
    wid = lax.axis_index("s") * NC + lax.axis_index("c")
    base = wid * B_PER_W
    lane = lax.iota(jnp.int32, L)
    half_mask = lane >= L // 2
    bufs = [bufs_flat[3 * t:3 * t + 3] for t in range(3)]
    idxs = (idx_h, idx_t, idx_r)

    def chunk_body(g, carry):
        cbase = base + g * CHUNK
        pltpu.sync_copy(heads_hbm.at[pl.ds(cbase, CHUNK)], idx_h)
        pltpu.sync_copy(tails_hbm.at[pl.ds(cbase, CHUNK)], idx_t)
        pltpu.sync_copy(rels_hbm.at[pl.ds(cbase, CHUNK)], idx_r)
        copies = []
        for t, table in enumerate((ent_hbm, ent_hbm, rel_hbm)):
            for k in range(3):
                copies.append(pltpu.async_copy(
                    table.at[idxs[t], pl.ds(128 * k, 128)], bufs[t][k], sem))
        for c in copies:
            c.wait()

        def body(i, carry2):
            acc = jnp.zeros((L,), jnp.float32)
            for re_p, re_o, im_p, im_o, masked in _GROUPS:
                h_re = bufs[0][re_p][i, pl.ds(re_o, L)]
                h_im = bufs[0][im_p][i, pl.ds(im_o, L)]
                t_re = bufs[1][re_p][i, pl.ds(re_o, L)]
                t_im = bufs[1][im_p][i, pl.ds(im_o, L)]
                r_re = bufs[2][re_p][i, pl.ds(re_o, L)]
                r_im = bufs[2][im_p][i, pl.ds(im_o, L)]
                p = h_re * t_re + h_im * t_im
                q = h_re * t_im - h_im * t_re
                term = r_re * p + r_im * q
                if masked:
                    term = jnp.where(half_mask, term, 0.0)
                acc = acc + term
            accbuf[pl.ds(i * ACC_STRIDE, L)] = acc
            return carry2

        lax.fori_loop(0, CHUNK, body, 0)
        _transpose_reduce(accbuf, out_v, g * CHUNK, lane, CHUNK)
        return carry

    lax.fori_loop(0, NCHUNK, chunk_body, 0)
    pltpu.sync_copy(out_v, out_hbm.at[pl.ds(base, B_PER_W)])


def _seg_kernel(heads_hbm, rels_hbm, tails_hbm, ent_seg_hbm, rel_seg_hbm,
                out_hbm, idx_h, idx_t, idx_r, rows_h, rows_t, rows_r,
                accbuf, out_v, sem):
    wid = lax.axis_index("s") * NC + lax.axis_index("c")
    base = wid * B_PER_W
    lane = lax.iota(jnp.int32, L)

    pltpu.sync_copy(heads_hbm.at[pl.ds(base, B_PER_W)], idx_h)
    pltpu.sync_copy(tails_hbm.at[pl.ds(base, B_PER_W)], idx_t)
    pltpu.sync_copy(rels_hbm.at[pl.ds(base, B_PER_W)], idx_r)
    c1 = pltpu.async_copy(ent_seg_hbm.at[idx_h], rows_h, sem)
    c2 = pltpu.async_copy(ent_seg_hbm.at[idx_t], rows_t, sem)
    c3 = pltpu.async_copy(rel_seg_hbm.at[idx_r], rows_r, sem)
    c1.wait()
    c2.wait()
    c3.wait()

    def body(i, carry):
        h_re = rows_h[i, pl.ds(0, L)]
        h_im = rows_h[i, pl.ds(L, L)]
        t_re = rows_t[i, pl.ds(0, L)]
        t_im = rows_t[i, pl.ds(L, L)]
        r_re = rows_r[i, pl.ds(0, L)]
        r_im = rows_r[i, pl.ds(L, L)]
        p = h_re * t_re + h_im * t_im
        q = h_re * t_im - h_im * t_re
        accbuf[pl.ds(i * ACC_STRIDE, L)] = r_re * p + r_im * q
        return carry

    lax.fori_loop(0, B_PER_W, body, 0)
    lane2 = lane
    for k in range(0, B_PER_W, L):
        base_idx = (lane2 + k) * ACC_STRIDE
        tot = jnp.zeros((L,), jnp.float32)
        for j in range(L):
            tot = tot + plsc.load_gather(accbuf, [base_idx + j])
        out_v[pl.ds(k, L)] = tot
    pltpu.sync_copy(out_v, out_hbm.at[pl.ds(base, B_PER_W)])


@jax.jit
def _compl_ex(heads, rels, tails, entity_emb, rel_emb):
    mesh = plsc.VectorSubcoreMesh(
        core_axis_name="c", subcore_axis_name="s", num_cores=NC,
        num_subcores=NS)
    main = functools.partial(
        pl.kernel,
        out_type=jax.ShapeDtypeStruct((BATCH,), jnp.float32),
        mesh=mesh,
        compiler_params=pltpu.CompilerParams(needs_layout_passes=False),
        scratch_types=[
            pltpu.VMEM((CHUNK,), jnp.int32),
            pltpu.VMEM((CHUNK,), jnp.int32),
            pltpu.VMEM((CHUNK,), jnp.int32),
            [pltpu.VMEM((CHUNK, 128), jnp.float32) for _ in range(9)],
            pltpu.VMEM((CHUNK * ACC_STRIDE,), jnp.float32),
            pltpu.VMEM((B_PER_W,), jnp.float32),
            pltpu.SemaphoreType.DMA,
        ],
    )(_main_kernel)
    seg = functools.partial(
        pl.kernel,
        out_type=jax.ShapeDtypeStruct((BATCH,), jnp.float32),
        mesh=mesh,
        compiler_params=pltpu.CompilerParams(
            needs_layout_passes=False, use_tc_tiling_on_sc=False),
        scratch_types=[
            pltpu.VMEM((B_PER_W,), jnp.int32),
            pltpu.VMEM((B_PER_W,), jnp.int32),
            pltpu.VMEM((B_PER_W,), jnp.int32),
            pltpu.VMEM((B_PER_W, 2 * L), jnp.float32),
            pltpu.VMEM((B_PER_W, 2 * L), jnp.float32),
            pltpu.VMEM((B_PER_W, 2 * L), jnp.float32),
            pltpu.VMEM((B_PER_W * ACC_STRIDE,), jnp.float32),
            pltpu.VMEM((B_PER_W,), jnp.float32),
            pltpu.SemaphoreType.DMA,
        ],
    )(_seg_kernel)
    ent_seg = jnp.concatenate(
        [entity_emb[:, SEG:DIM], entity_emb[:, DIM + SEG:]], axis=1)
    rel_seg = jnp.concatenate(
        [rel_emb[:, SEG:DIM], rel_emb[:, DIM + SEG:]], axis=1)
    part1 = main(heads, rels, tails, entity_emb, rel_emb)
    part2 = seg(heads, rels, tails, ent_seg, rel_seg)
    return part1 + part2


def kernel(heads, rels, tails, entity_emb, rel_emb):
    return _compl_ex(
        heads.astype(jnp.int32),
        rels.astype(jnp.int32),
        tails.astype(jnp.int32),
        entity_emb.astype(jnp.float32),
        rel_emb.astype(jnp.float32),
    )
